# baseline (device time: 1440421 ns/iter reference)
import jax
import jax.numpy as jnp
from jax import lax
from jax.experimental import pallas as pl
from jax.experimental.pallas import tpu as pltpu

N_DEV = 16


def kernel(x, w_mat):
    m, k_per = x.shape
    _, n = w_mat.shape
    m_per = m // N_DEV

    x = x.astype(jnp.bfloat16)
    w_mat = w_mat.astype(jnp.bfloat16)

    def body(x_ref, w_ref, out_ref, comm_ref, send_sems, recv_sems, credit_sem):
        my = lax.axis_index("i")
        left = lax.rem(my + N_DEV - 1, N_DEV)
        right = lax.rem(my + 1, N_DEV)

        barrier = pltpu.get_barrier_semaphore()
        for nbr in (left, right):
            pl.semaphore_signal(
                barrier, inc=1,
                device_id=(nbr,), device_id_type=pl.DeviceIdType.MESH,
            )
        pl.semaphore_wait(barrier, 2)

        def partial_for(dst):
            rows = x_ref[pl.ds(dst * m_per, m_per), :]
            return jnp.dot(rows, w_ref[:, :], preferred_element_type=jnp.float32)

        for s in range(N_DEV - 1):
            send_slot = s % 2
            recv_slot = (s + 1) % 2
            dst = lax.rem(my + (2 * N_DEV - 1 - s), N_DEV)
            p = partial_for(dst)
            if s == 0:
                acc = p
            else:
                acc = p + comm_ref[send_slot]
            comm_ref[send_slot] = acc
            if s >= 1:
                pl.semaphore_wait(credit_sem, 1)
            rdma = pltpu.make_async_remote_copy(
                src_ref=comm_ref.at[send_slot],
                dst_ref=comm_ref.at[recv_slot],
                send_sem=send_sems.at[send_slot],
                recv_sem=recv_sems.at[recv_slot],
                device_id=(right,),
                device_id_type=pl.DeviceIdType.MESH,
            )
            rdma.start()
            rdma.wait()
            if s < N_DEV - 2:
                pl.semaphore_signal(
                    credit_sem, inc=1,
                    device_id=(left,), device_id_type=pl.DeviceIdType.MESH,
                )

        out_ref[:, :] = partial_for(my) + comm_ref[(N_DEV - 1) % 2]

    return pl.pallas_call(
        body,
        out_shape=jax.ShapeDtypeStruct((m_per, n), jnp.float32),
        in_specs=[
            pl.BlockSpec(memory_space=pltpu.VMEM),
            pl.BlockSpec(memory_space=pltpu.VMEM),
        ],
        out_specs=pl.BlockSpec(memory_space=pltpu.VMEM),
        scratch_shapes=[
            pltpu.VMEM((2, m_per, n), jnp.float32),
            pltpu.SemaphoreType.DMA((2,)),
            pltpu.SemaphoreType.DMA((2,)),
            pltpu.SemaphoreType.REGULAR,
        ],
        compiler_params=pltpu.CompilerParams(collective_id=0),
    )(x, w_mat)


# device time: 429461 ns/iter; 3.3540x vs baseline; 3.3540x over previous
import jax
import jax.numpy as jnp
from jax import lax
from jax.experimental import pallas as pl
from jax.experimental.pallas import tpu as pltpu

N_DEV = 16


def kernel(x, w_mat):
    m, k_per = x.shape
    _, n = w_mat.shape
    m_per = m // N_DEV
    nh = n // 2

    x = x.astype(jnp.bfloat16)
    w_mat = w_mat.astype(jnp.bfloat16)

    def body(x_ref, w_ref, out_ref, cw_ref, ccw_ref,
             cw_send_sems, cw_recv_sems, ccw_send_sems, ccw_recv_sems,
             cw_credit, ccw_credit):
        my = lax.axis_index("i")
        left = lax.rem(my + N_DEV - 1, N_DEV)
        right = lax.rem(my + 1, N_DEV)

        barrier = pltpu.get_barrier_semaphore()
        for nbr in (left, right):
            pl.semaphore_signal(
                barrier, inc=1,
                device_id=(nbr,), device_id_type=pl.DeviceIdType.MESH,
            )
        pl.semaphore_wait(barrier, 2)

        def pcw(dst):
            rows = x_ref[pl.ds(dst * m_per, m_per), :]
            return jnp.dot(rows, w_ref[:, :nh], preferred_element_type=jnp.float32)

        def pccw(dst):
            rows = x_ref[pl.ds(dst * m_per, m_per), :]
            return jnp.dot(rows, w_ref[:, nh:], preferred_element_type=jnp.float32)

        def dst_cw(s):
            return lax.rem(my + (2 * N_DEV - 1 - s), N_DEV)

        def dst_ccw(s):
            return lax.rem(my + 1 + s, N_DEV)

        p_cw = pcw(dst_cw(0))
        p_ccw = pccw(dst_ccw(0))
        for s in range(N_DEV - 1):
            ss = s % 2
            rs = (s + 1) % 2
            if s == 0:
                acc_cw, acc_ccw = p_cw, p_ccw
            else:
                acc_cw = p_cw + cw_ref[ss].astype(jnp.float32)
                acc_ccw = p_ccw + ccw_ref[ss].astype(jnp.float32)
            cw_ref[ss] = acc_cw.astype(jnp.bfloat16)
            ccw_ref[ss] = acc_ccw.astype(jnp.bfloat16)
            if s >= 1:
                pl.semaphore_wait(cw_credit, 1)
                pl.semaphore_wait(ccw_credit, 1)
            r_cw = pltpu.make_async_remote_copy(
                src_ref=cw_ref.at[ss],
                dst_ref=cw_ref.at[rs],
                send_sem=cw_send_sems.at[ss],
                recv_sem=cw_recv_sems.at[rs],
                device_id=(right,),
                device_id_type=pl.DeviceIdType.MESH,
            )
            r_ccw = pltpu.make_async_remote_copy(
                src_ref=ccw_ref.at[ss],
                dst_ref=ccw_ref.at[rs],
                send_sem=ccw_send_sems.at[ss],
                recv_sem=ccw_recv_sems.at[rs],
                device_id=(left,),
                device_id_type=pl.DeviceIdType.MESH,
            )
            r_cw.start()
            r_ccw.start()
            p_cw = pcw(dst_cw(s + 1))
            p_ccw = pccw(dst_ccw(s + 1))
            r_cw.wait()
            r_ccw.wait()
            if s < N_DEV - 2:
                pl.semaphore_signal(
                    cw_credit, inc=1,
                    device_id=(left,), device_id_type=pl.DeviceIdType.MESH,
                )
                pl.semaphore_signal(
                    ccw_credit, inc=1,
                    device_id=(right,), device_id_type=pl.DeviceIdType.MESH,
                )

        out_ref[:, :nh] = p_cw + cw_ref[1].astype(jnp.float32)
        out_ref[:, nh:] = p_ccw + ccw_ref[1].astype(jnp.float32)

    return pl.pallas_call(
        body,
        out_shape=jax.ShapeDtypeStruct((m_per, n), jnp.float32),
        in_specs=[
            pl.BlockSpec(memory_space=pltpu.VMEM),
            pl.BlockSpec(memory_space=pltpu.VMEM),
        ],
        out_specs=pl.BlockSpec(memory_space=pltpu.VMEM),
        scratch_shapes=[
            pltpu.VMEM((2, m_per, nh), jnp.bfloat16),
            pltpu.VMEM((2, m_per, nh), jnp.bfloat16),
            pltpu.SemaphoreType.DMA((2,)),
            pltpu.SemaphoreType.DMA((2,)),
            pltpu.SemaphoreType.DMA((2,)),
            pltpu.SemaphoreType.DMA((2,)),
            pltpu.SemaphoreType.REGULAR,
            pltpu.SemaphoreType.REGULAR,
        ],
        compiler_params=pltpu.CompilerParams(collective_id=0),
    )(x, w_mat)


# device time: 355745 ns/iter; 4.0490x vs baseline; 1.2072x over previous
import jax
import jax.numpy as jnp
from jax import lax
from jax.experimental import pallas as pl
from jax.experimental.pallas import tpu as pltpu

N_DEV = 16
N_SLOT = 3


def kernel(x, w_mat):
    m, k_per = x.shape
    _, n = w_mat.shape
    m_per = m // N_DEV
    nh = n // 2
    nq = nh // 2

    x = x.astype(jnp.bfloat16)
    w_mat = w_mat.astype(jnp.bfloat16)

    def body(x_ref, w_ref, out_ref, comm_ref, send_sems, recv_sems,
             cred0, cred1, cred2, cred3):
        creds = [cred0, cred1, cred2, cred3]
        my = lax.axis_index("i")
        left = lax.rem(my + N_DEV - 1, N_DEV)
        right = lax.rem(my + 1, N_DEV)

        barrier = pltpu.get_barrier_semaphore()
        for nbr in (left, right):
            pl.semaphore_signal(
                barrier, inc=1,
                device_id=(nbr,), device_id_type=pl.DeviceIdType.MESH,
            )
        pl.semaphore_wait(barrier, 2)

        ring_cols = {0: 0, 1: nh, 2: nq, 3: nh + nq}
        ring_tgt = {0: right, 1: left, 2: right, 3: left}
        ring_up = {0: left, 1: right, 2: left, 3: right}

        def partials(s):
            d_cw = lax.rem(my + (2 * N_DEV - 1 - s), N_DEV)
            d_ccw = lax.rem(my + 1 + s, N_DEV)
            p_cw = jnp.dot(
                x_ref[pl.ds(d_cw * m_per, m_per), :], w_ref[:, :nh],
                preferred_element_type=jnp.float32,
            )
            p_ccw = jnp.dot(
                x_ref[pl.ds(d_ccw * m_per, m_per), :], w_ref[:, nh:],
                preferred_element_type=jnp.float32,
            )
            return p_cw, p_ccw

        def ring_p(p_cw, p_ccw, r):
            half = p_cw if r in (0, 2) else p_ccw
            c = 0 if r in (0, 1) else nq
            return half[:, c:c + nq]

        def mk(r, s):
            return pltpu.make_async_remote_copy(
                src_ref=comm_ref.at[r * N_SLOT + s % N_SLOT],
                dst_ref=comm_ref.at[r * N_SLOT + (s + 1) % N_SLOT],
                send_sem=send_sems.at[r * N_SLOT + s % N_SLOT],
                recv_sem=recv_sems.at[r * N_SLOT + (s + 1) % N_SLOT],
                device_id=(ring_tgt[r],),
                device_id_type=pl.DeviceIdType.MESH,
            )

        p_cw, p_ccw = partials(0)
        for s in range(N_DEV - 1):
            for r in range(4):
                if s >= 1:
                    prev = mk(r, s - 1)
                    prev.wait_send()
                    if s <= N_DEV - 3:
                        pl.semaphore_signal(
                            creds[r], inc=1,
                            device_id=(ring_up[r],),
                            device_id_type=pl.DeviceIdType.MESH,
                        )
                    prev.wait_recv()
                p = ring_p(p_cw, p_ccw, r)
                slot = r * N_SLOT + s % N_SLOT
                if s == 0:
                    acc = p
                else:
                    acc = p + comm_ref[slot].astype(jnp.float32)
                comm_ref[slot] = acc.astype(jnp.bfloat16)
                if s >= 2:
                    pl.semaphore_wait(creds[r], 1)
                mk(r, s).start()
            p_cw, p_ccw = partials(s + 1)

        for r in range(4):
            fin = mk(r, N_DEV - 2)
            fin.wait_send()
            fin.wait_recv()
            c = ring_cols[r]
            out_ref[:, c:c + nq] = (
                ring_p(p_cw, p_ccw, r) + comm_ref[r * N_SLOT].astype(jnp.float32)
            )

    return pl.pallas_call(
        body,
        out_shape=jax.ShapeDtypeStruct((m_per, n), jnp.float32),
        in_specs=[
            pl.BlockSpec(memory_space=pltpu.VMEM),
            pl.BlockSpec(memory_space=pltpu.VMEM),
        ],
        out_specs=pl.BlockSpec(memory_space=pltpu.VMEM),
        scratch_shapes=[
            pltpu.VMEM((4 * N_SLOT, m_per, nq), jnp.bfloat16),
            pltpu.SemaphoreType.DMA((4 * N_SLOT,)),
            pltpu.SemaphoreType.DMA((4 * N_SLOT,)),
            pltpu.SemaphoreType.REGULAR,
            pltpu.SemaphoreType.REGULAR,
            pltpu.SemaphoreType.REGULAR,
            pltpu.SemaphoreType.REGULAR,
        ],
        compiler_params=pltpu.CompilerParams(collective_id=0),
    )(x, w_mat)
